# baseline (speedup 1.0000x reference)
import functools

import jax
import jax.numpy as jnp
from jax import lax
from jax.experimental import pallas as pl
from jax.experimental.pallas import tpu as pltpu
from jax.experimental.pallas import tpu_sc as plsc

NUM_EXPERTS = 8
TOP_K = 2
D_IN = 1024
D_FF = 2048
D_OUT = 1024
T = 2048

BM = 128
NT = 39
R_MM = NT * BM
R = 5120


def _gating_body(x_ref, wg_ref, bg_ref, w_ref, i_ref):
    logits = jnp.dot(x_ref[...], wg_ref[...], preferred_element_type=jnp.float32)
    logits = logits + bg_ref[...]
    m = jnp.max(logits, axis=1, keepdims=True)
    p = jnp.exp(logits - m)
    g = p / jnp.sum(p, axis=1, keepdims=True)
    iota = lax.broadcasted_iota(jnp.int32, g.shape, 1)
    w1 = jnp.max(g, axis=1, keepdims=True)
    i1 = jnp.min(jnp.where(g == w1, iota, NUM_EXPERTS), axis=1, keepdims=True)
    g2 = jnp.where(iota == i1, -1.0, g)
    w2 = jnp.max(g2, axis=1, keepdims=True)
    i2 = jnp.min(jnp.where(g2 == w2, iota, NUM_EXPERTS), axis=1, keepdims=True)
    w_ref[...] = jnp.concatenate([w1, w2], axis=1)
    i_ref[...] = jnp.concatenate([i1, i2], axis=1)


def _gating(x, Wg, bg):
    return pl.pallas_call(
        _gating_body,
        out_shape=[
            jax.ShapeDtypeStruct((T, TOP_K), jnp.float32),
            jax.ShapeDtypeStruct((T, TOP_K), jnp.int32),
        ],
    )(x, Wg, bg.reshape(1, NUM_EXPERTS))


def _gather_rows(x, row_tok):
    info = plsc.get_sparse_core_info()
    nw = info.num_cores * info.num_subcores
    rows_pw = R // nw
    chunk = rows_pw // 2
    mesh = plsc.VectorSubcoreMesh(core_axis_name="c", subcore_axis_name="s")

    @functools.partial(
        pl.kernel, mesh=mesh,
        out_type=jax.ShapeDtypeStruct((R, D_IN), jnp.float32),
        scratch_types=[
            pltpu.VMEM((chunk,), jnp.int32),
            pltpu.VMEM((chunk, D_IN), jnp.float32),
            pltpu.SemaphoreType.DMA,
        ],
    )
    def k(x_hbm, tok_hbm, out_hbm, idx_v, rows_v, sem):
        wid = lax.axis_index("s") * info.num_cores + lax.axis_index("c")
        for ci in range(2):
            base = wid * rows_pw + ci * chunk
            pltpu.sync_copy(tok_hbm.at[pl.ds(base, chunk)], idx_v)
            pltpu.async_copy(x_hbm.at[idx_v], rows_v, sem).wait()
            pltpu.sync_copy(rows_v, out_hbm.at[pl.ds(base, chunk)])

    return k(x, row_tok)


def _ffn_body(te_ref, xs_ref, w1_ref, b1_ref, w2_ref, b2_ref, rw_ref, out_ref):
    h = jnp.dot(xs_ref[...], w1_ref[0], preferred_element_type=jnp.float32)
    h = jnp.maximum(h + b1_ref[0], 0.0)
    y = jnp.dot(h, w2_ref[0], preferred_element_type=jnp.float32)
    y = y + b2_ref[0]
    out_ref[...] = y * rw_ref[...]


def _expert_ffn(xs, tile_e, row_w, W1, b1, W2, b2):
    grid_spec = pltpu.PrefetchScalarGridSpec(
        num_scalar_prefetch=1,
        grid=(NT,),
        in_specs=[
            pl.BlockSpec((BM, D_IN), lambda i, te: (i, 0)),
            pl.BlockSpec((1, D_IN, D_FF), lambda i, te: (te[i], 0, 0)),
            pl.BlockSpec((1, 1, D_FF), lambda i, te: (te[i], 0, 0)),
            pl.BlockSpec((1, D_FF, D_OUT), lambda i, te: (te[i], 0, 0)),
            pl.BlockSpec((1, 1, D_OUT), lambda i, te: (te[i], 0, 0)),
            pl.BlockSpec((BM, 1), lambda i, te: (i, 0)),
        ],
        out_specs=pl.BlockSpec((BM, D_OUT), lambda i, te: (i, 0)),
    )
    return pl.pallas_call(
        _ffn_body,
        grid_spec=grid_spec,
        out_shape=jax.ShapeDtypeStruct((R_MM, D_OUT), jnp.float32),
        compiler_params=pltpu.CompilerParams(
            dimension_semantics=("arbitrary",)),
    )(tile_e, xs, W1, b1, W2, b2, row_w)


def _combine(ys, pos0, pos1):
    info = plsc.get_sparse_core_info()
    nw = info.num_cores * info.num_subcores
    toks_pw = T // nw
    chunk = toks_pw // 2
    mesh = plsc.VectorSubcoreMesh(core_axis_name="c", subcore_axis_name="s")

    @functools.partial(
        pl.kernel, mesh=mesh,
        out_type=jax.ShapeDtypeStruct((T, D_OUT), jnp.float32),
        scratch_types=[
            pltpu.VMEM((chunk,), jnp.int32),
            pltpu.VMEM((chunk,), jnp.int32),
            pltpu.VMEM((chunk, D_OUT), jnp.float32),
            pltpu.VMEM((chunk, D_OUT), jnp.float32),
            pltpu.SemaphoreType.DMA,
        ],
    )
    def k(ys_hbm, p0_hbm, p1_hbm, out_hbm, i0_v, i1_v, r0_v, r1_v, sem):
        wid = lax.axis_index("s") * info.num_cores + lax.axis_index("c")
        for ci in range(2):
            base = wid * toks_pw + ci * chunk
            pltpu.sync_copy(p0_hbm.at[pl.ds(base, chunk)], i0_v)
            pltpu.sync_copy(p1_hbm.at[pl.ds(base, chunk)], i1_v)
            pltpu.async_copy(ys_hbm.at[i0_v], r0_v, sem).wait()
            pltpu.async_copy(ys_hbm.at[i1_v], r1_v, sem).wait()

            def row_add(r, _):
                for c in range(D_OUT // 16):
                    sl = pl.ds(c * 16, 16)
                    r0_v[r, sl] = r0_v[r, sl] + r1_v[r, sl]
                return 0

            lax.fori_loop(0, chunk, row_add, 0)
            pltpu.sync_copy(r0_v, out_hbm.at[pl.ds(base, chunk)])

    return k(ys, pos0, pos1)


def kernel(x, Wg, bg, W1, b1, W2, b2):
    w, eidx = _gating(x, Wg, bg)

    flat_e = eidx.reshape(-1)
    flat_w = w.reshape(-1)
    flat_tok = jnp.arange(T * TOP_K, dtype=jnp.int32) // TOP_K
    order = jnp.argsort(flat_e)
    sorted_e = flat_e[order]
    counts = jnp.sum(
        (flat_e[:, None] == jnp.arange(NUM_EXPERTS, dtype=flat_e.dtype)[None, :]
         ).astype(jnp.int32), axis=0)
    padded = ((counts + BM - 1) // BM) * BM
    seg_end = jnp.cumsum(padded)
    seg_start = seg_end - padded
    sort_start = jnp.cumsum(counts) - counts
    rank = jnp.arange(T * TOP_K, dtype=jnp.int32) - sort_start[sorted_e]
    dst = (seg_start[sorted_e] + rank).astype(jnp.int32)
    row_tok = jnp.zeros((R,), jnp.int32).at[dst].set(flat_tok[order])
    row_w = jnp.zeros((R,), jnp.float32).at[dst].set(flat_w[order])
    pos = jnp.zeros((T * TOP_K,), jnp.int32).at[order].set(dst).reshape(T, TOP_K)
    tile_start = jnp.arange(NT, dtype=jnp.int32) * BM
    tile_e = jnp.sum((tile_start[:, None] >= seg_end[None, :]).astype(jnp.int32),
                     axis=1)
    tile_e = jnp.minimum(tile_e, NUM_EXPERTS - 1).astype(jnp.int32)

    xs = _gather_rows(x, row_tok)
    ys = _expert_ffn(xs[:R_MM], tile_e, row_w[:R_MM, None],
                     W1, b1.reshape(NUM_EXPERTS, 1, D_FF),
                     W2, b2.reshape(NUM_EXPERTS, 1, D_OUT))
    out = _combine(ys, pos[:, 0], pos[:, 1])
    return out
